# baseline (device time: 14022 ns/iter reference)
import jax
import jax.numpy as jnp
from jax import lax
from jax.experimental import pallas as pl
from jax.experimental.pallas import tpu as pltpu


def kernel(x, W, labels):
    T, D = x.shape
    V = W.shape[1]
    labels2 = labels.reshape(T, 1)

    def body(x_ref, w_ref, lab_ref, out_ref, send_buf, recv_buf,
             send_sem, recv_sem):
        my_x = lax.axis_index("x")
        my_y = lax.axis_index("y")
        my_z = lax.axis_index("z")
        partner = (1 - my_x, my_y, my_z)

        logits = jnp.dot(
            x_ref[:, :].astype(jnp.bfloat16),
            w_ref[:, :].astype(jnp.bfloat16),
            preferred_element_type=jnp.float32,
        )

        m = jnp.max(logits, axis=1, keepdims=True)
        s = jnp.sum(jnp.exp(logits - m), axis=1, keepdims=True)
        col = lax.broadcasted_iota(jnp.int32, (T, V), 1)
        lab_local = lab_ref[:, :] - my_x * V
        c = jnp.sum(jnp.where(col == lab_local, logits, 0.0),
                    axis=1, keepdims=True)

        send_buf[:, 0:1] = m
        send_buf[:, 1:2] = s
        send_buf[:, 2:3] = c

        barrier = pltpu.get_barrier_semaphore()
        pl.semaphore_signal(barrier, inc=1, device_id=partner,
                            device_id_type=pl.DeviceIdType.MESH)
        pl.semaphore_wait(barrier, 1)

        rdma = pltpu.make_async_remote_copy(
            src_ref=send_buf,
            dst_ref=recv_buf,
            send_sem=send_sem,
            recv_sem=recv_sem,
            device_id=partner,
            device_id_type=pl.DeviceIdType.MESH,
        )
        rdma.start()
        rdma.wait()

        mo = recv_buf[:, 0:1]
        so = recv_buf[:, 1:2]
        co = recv_buf[:, 2:3]
        gm = jnp.maximum(m, mo)
        gs = s * jnp.exp(m - gm) + so * jnp.exp(mo - gm)
        out_ref[:, :] = gm + jnp.log(gs) - (c + co)

    out = pl.pallas_call(
        body,
        out_shape=jax.ShapeDtypeStruct((T, 1), jnp.float32),
        in_specs=[
            pl.BlockSpec(memory_space=pltpu.VMEM),
            pl.BlockSpec(memory_space=pltpu.VMEM),
            pl.BlockSpec(memory_space=pltpu.VMEM),
        ],
        out_specs=pl.BlockSpec(memory_space=pltpu.VMEM),
        scratch_shapes=[
            pltpu.VMEM((T, 4), jnp.float32),
            pltpu.VMEM((T, 4), jnp.float32),
            pltpu.SemaphoreType.DMA,
            pltpu.SemaphoreType.DMA,
        ],
        compiler_params=pltpu.CompilerParams(collective_id=0),
    )(x, W, labels2)
    return out[:, 0]
